# baseline (speedup 1.0000x reference)
import functools

import jax
import jax.numpy as jnp
from jax import lax
from jax.experimental import pallas as pl
from jax.experimental.pallas import tpu as pltpu
from jax.experimental.pallas import tpu_sc as plsc

V, D, B, C = 1000000, 128, 16384, 20
NC, NS, L = 2, 16, 16
NW = NC * NS
BPW = B // NW
CH = 8
NCHUNK = BPW // CH
RPC = CH * C
IW = 80
NIR = RPC // IW
DK = D // L
NBUF = 4


def _sc_kernel(tgt_idx_hbm, ctx_idx_hbm, tgt_table, ctx_table, out_hbm,
               tidx_v, craw_v, cidx_v, trows, crows, out_v, sems):
  wid = lax.axis_index("s") * NC + lax.axis_index("c")

  lane = lax.iota(jnp.int32, L)
  xor_idx = [lane ^ s for s in (8, 4, 2, 1)]

  pltpu.sync_copy(tgt_idx_hbm.at[pl.ds(wid * BPW, BPW)], tidx_v)
  for c in range(C):
    pltpu.async_copy(ctx_idx_hbm.at[pl.ds(c * B + wid * BPW, BPW)],
                     craw_v.at[c], sems[0])
  for c in range(C):
    pltpu.make_async_copy(ctx_idx_hbm.at[pl.ds(c * B + wid * BPW, BPW)],
                          craw_v.at[c], sems[0]).wait()

  lane_c = lane * C

  @plsc.parallel_loop(0, BPW // L, unroll=2)
  def repack(w):
    for c in range(C):
      v = craw_v[c, pl.ds(w * L, L)]
      plsc.store_scatter(cidx_v, [lane_c + (w * L * C + c)], v)

  def issue(j, q):
    pltpu.async_copy(tgt_table.at[tidx_v.at[pl.ds(j * CH, CH)]], trows[q],
                     sems[q])
    for k in range(NIR):
      pltpu.async_copy(ctx_table.at[cidx_v.at[pl.ds(j * RPC + k * IW, IW)]],
                       crows[q].at[pl.ds(k * IW, IW)], sems[q])

  def drain(j, q):
    pltpu.make_async_copy(tgt_table.at[tidx_v.at[pl.ds(j * CH, CH)]],
                          trows[q], sems[q]).wait()
    for k in range(NIR):
      pltpu.make_async_copy(
          ctx_table.at[cidx_v.at[pl.ds(j * RPC + k * IW, IW)]],
          crows[q].at[pl.ds(k * IW, IW)], sems[q]).wait()

  lane_b = lane * BPW

  def compute(j, q):
    trow_v = trows[q]
    crow_v = crows[q]

    @plsc.parallel_loop(0, CH, unroll=2)
    def tgt_body(i):
      t = [trow_v[i, pl.ds(kk * L, L)] for kk in range(DK)]
      b_local = j * CH + i
      res1 = jnp.zeros((L,), jnp.float32)
      res2 = jnp.zeros((L,), jnp.float32)
      for c in range(C):
        r = i * C + c
        p = [t[kk] * crow_v[r, pl.ds(kk * L, L)] for kk in range(DK)]
        acc = ((p[0] + p[1]) + (p[2] + p[3])) + ((p[4] + p[5]) + (p[6] + p[7]))
        for x in xor_idx:
          acc = acc + acc.at[x].get(mode="promise_in_bounds")
        if c < L:
          res1 = jnp.where(lane == c, acc, res1)
        else:
          res2 = jnp.where(lane == c - L, acc, res2)
      plsc.store_scatter(out_v, [lane_b + b_local], res1)
      plsc.store_scatter(out_v, [lane_b + (L * BPW + b_local)], res2,
                         mask=lane < C - L)

  for q in range(NBUF - 1):
    issue(q, q)

  def chunk_group(jj, carry):
    for q in range(NBUF):
      j = NBUF * jj + q
      drain(j, q)
      compute(j, q)

      @pl.when(j + NBUF - 1 < NCHUNK)
      def _():
        issue(j + NBUF - 1, (q + NBUF - 1) % NBUF)
    return carry

  lax.fori_loop(0, NCHUNK // NBUF, chunk_group, 0)

  for c in range(C):
    pltpu.async_copy(out_v.at[pl.ds(c * BPW, BPW)],
                     out_hbm.at[pl.ds(c * B + wid * BPW, BPW)], sems[0])
  for c in range(C):
    pltpu.make_async_copy(out_v.at[pl.ds(c * BPW, BPW)],
                          out_hbm.at[pl.ds(c * B + wid * BPW, BPW)],
                          sems[0]).wait()


@jax.jit
def _run(tgt_idx, ctx_idx, tgt_table, ctx_table):
  def body(tgt_idx_hbm, ctx_idx_hbm, tgt_table_hbm, ctx_table_hbm, out_hbm,
           tidx_v, craw_v, cidx_v,
           tr0, cr0, tr1, cr1, tr2, cr2, tr3, cr3, out_v, s0, s1, s2, s3):
    _sc_kernel(tgt_idx_hbm, ctx_idx_hbm, tgt_table_hbm, ctx_table_hbm,
               out_hbm, tidx_v, craw_v, cidx_v,
               [tr0, tr1, tr2, tr3], [cr0, cr1, cr2, cr3], out_v,
               [s0, s1, s2, s3])

  kfn = pl.kernel(
      body,
      out_type=jax.ShapeDtypeStruct((B * C,), jnp.float32),
      mesh=plsc.VectorSubcoreMesh(core_axis_name="c", subcore_axis_name="s"),
      compiler_params=pltpu.CompilerParams(needs_layout_passes=False),
      scratch_types=[
          pltpu.VMEM((BPW,), jnp.int32),
          pltpu.VMEM((C, BPW), jnp.int32),
          pltpu.VMEM((BPW * C,), jnp.int32),
          pltpu.VMEM((CH, D), jnp.float32),
          pltpu.VMEM((RPC, D), jnp.float32),
          pltpu.VMEM((CH, D), jnp.float32),
          pltpu.VMEM((RPC, D), jnp.float32),
          pltpu.VMEM((CH, D), jnp.float32),
          pltpu.VMEM((RPC, D), jnp.float32),
          pltpu.VMEM((CH, D), jnp.float32),
          pltpu.VMEM((RPC, D), jnp.float32),
          pltpu.VMEM((BPW * C,), jnp.float32),
          pltpu.SemaphoreType.DMA,
          pltpu.SemaphoreType.DMA,
          pltpu.SemaphoreType.DMA,
          pltpu.SemaphoreType.DMA,
      ],
  )
  return kfn(tgt_idx, ctx_idx, tgt_table, ctx_table)


def kernel(target, context, tgt_table, ctx_table):
  tgt_idx = target.astype(jnp.int32)
  ctx_idx = context.astype(jnp.int32).T.reshape(C * B)
  out = _run(tgt_idx, ctx_idx, tgt_table, ctx_table)
  return out.reshape(C, B).T
